# baseline (speedup 1.0000x reference)
import functools

import jax
import jax.numpy as jnp
from jax import lax
from jax.experimental import pallas as pl
from jax.experimental.pallas import tpu as pltpu
from jax.experimental.pallas import tpu_sc as plsc

NC = 2
NS = 16
NW = NC * NS
CHUNK = 128
NSLOT = 6
AHEAD = 4


def _body(nchunk, table_hbm, idx_hbm, out_hbm, idx_v, rows_v, *sems):
    gsem, wsem = sems[:NSLOT], sems[NSLOT:]
    wid = lax.axis_index("s") * NC + lax.axis_index("c")
    pltpu.sync_copy(idx_hbm.at[wid], idx_v)

    def gather_descriptor(j, slot):
        return pltpu.make_async_copy(
            table_hbm.at[idx_v.at[j]], rows_v.at[slot], gsem[slot]
        )

    def write_descriptor(j, slot):
        return pltpu.make_async_copy(
            rows_v.at[slot], out_hbm.at[wid, j], wsem[slot]
        )

    for j in range(AHEAD):
        gather_descriptor(j, j).start()

    @pl.loop(0, nchunk, step=NSLOT)
    def _chunks(j0):
        for slot in range(NSLOT):
            j = j0 + slot

            @pl.when(j < nchunk)
            def _do_chunk():
                gather_descriptor(j, slot).wait()
                write_descriptor(j, slot).start()
                nslot = (slot + AHEAD) % NSLOT

                @pl.when(j + AHEAD < nchunk)
                def _prefetch():
                    @pl.when(j >= NSLOT - AHEAD)
                    def _free_slot():
                        write_descriptor(j + AHEAD - NSLOT, nslot).wait()

                    gather_descriptor(j + AHEAD, nslot).start()

    for j in range(nchunk):
        if j + NSLOT >= nchunk:
            write_descriptor(j, j % NSLOT).wait()


def kernel(tags, tag_embedding_weight):
    B_total = tags.shape[0] * tags.shape[1]
    D = tag_embedding_weight.shape[1]
    assert B_total % (NW * CHUNK) == 0
    nchunk = B_total // (NW * CHUNK)
    assert nchunk >= NSLOT > AHEAD

    idx = tags.reshape(NW, nchunk, CHUNK).astype(jnp.int32)

    mesh = plsc.VectorSubcoreMesh(
        core_axis_name="c", subcore_axis_name="s", num_cores=NC, num_subcores=NS
    )
    out = pl.kernel(
        functools.partial(_body, nchunk),
        out_type=jax.ShapeDtypeStruct((NW, nchunk, CHUNK, D), jnp.float32),
        mesh=mesh,
        scratch_types=[
            pltpu.VMEM((nchunk, CHUNK), jnp.int32),
            pltpu.VMEM((NSLOT, CHUNK, D), jnp.float32),
        ] + [pltpu.SemaphoreType.DMA] * (2 * NSLOT),
    )(tag_embedding_weight, idx)
    return out.reshape(tags.shape[0], tags.shape[1], D)


# speedup vs baseline: 1.0032x; 1.0032x over previous
import functools

import jax
import jax.numpy as jnp
from jax import lax
from jax.experimental import pallas as pl
from jax.experimental.pallas import tpu as pltpu
from jax.experimental.pallas import tpu_sc as plsc

NC = 2
NS = 16
NW = NC * NS
CHUNK = 128
NSLOT = 7
AHEAD = 4


def _body(nchunk, table_hbm, idx_hbm, out_hbm, idx_v, rows_v, *sems):
    gsem, wsem = sems[:NSLOT], sems[NSLOT:]
    wid = lax.axis_index("s") * NC + lax.axis_index("c")
    pltpu.sync_copy(idx_hbm.at[wid], idx_v)

    def gather_descriptor(j, slot):
        return pltpu.make_async_copy(
            table_hbm.at[idx_v.at[j]], rows_v.at[slot], gsem[slot]
        )

    def write_descriptor(j, slot):
        return pltpu.make_async_copy(
            rows_v.at[slot], out_hbm.at[wid, j], wsem[slot]
        )

    for j in range(AHEAD):
        gather_descriptor(j, j).start()

    @pl.loop(0, nchunk, step=NSLOT)
    def _chunks(j0):
        for slot in range(NSLOT):
            j = j0 + slot

            @pl.when(j < nchunk)
            def _do_chunk():
                gather_descriptor(j, slot).wait()
                write_descriptor(j, slot).start()
                nslot = (slot + AHEAD) % NSLOT

                @pl.when(j + AHEAD < nchunk)
                def _prefetch():
                    @pl.when(j >= NSLOT - AHEAD)
                    def _free_slot():
                        write_descriptor(j + AHEAD - NSLOT, nslot).wait()

                    gather_descriptor(j + AHEAD, nslot).start()

    for j in range(nchunk):
        if j + NSLOT >= nchunk:
            write_descriptor(j, j % NSLOT).wait()


def kernel(tags, tag_embedding_weight):
    B_total = tags.shape[0] * tags.shape[1]
    D = tag_embedding_weight.shape[1]
    assert B_total % (NW * CHUNK) == 0
    nchunk = B_total // (NW * CHUNK)
    assert nchunk >= NSLOT > AHEAD

    idx = tags.reshape(NW, nchunk, CHUNK).astype(jnp.int32)

    mesh = plsc.VectorSubcoreMesh(
        core_axis_name="c", subcore_axis_name="s", num_cores=NC, num_subcores=NS
    )
    out = pl.kernel(
        functools.partial(_body, nchunk),
        out_type=jax.ShapeDtypeStruct((NW, nchunk, CHUNK, D), jnp.float32),
        mesh=mesh,
        scratch_types=[
            pltpu.VMEM((nchunk, CHUNK), jnp.int32),
            pltpu.VMEM((NSLOT, CHUNK, D), jnp.float32),
        ] + [pltpu.SemaphoreType.DMA] * (2 * NSLOT),
    )(tag_embedding_weight, idx)
    return out.reshape(tags.shape[0], tags.shape[1], D)


# speedup vs baseline: 1.0081x; 1.0049x over previous
import functools

import jax
import jax.numpy as jnp
from jax import lax
from jax.experimental import pallas as pl
from jax.experimental.pallas import tpu as pltpu
from jax.experimental.pallas import tpu_sc as plsc

NC = 2
NS = 16
NW = NC * NS
CHUNK = 128
NSLOT = 7
AHEAD = 5


def _body(nchunk, table_hbm, idx_hbm, out_hbm, idx_v, rows_v, *sems):
    gsem, wsem = sems[:NSLOT], sems[NSLOT:]
    wid = lax.axis_index("s") * NC + lax.axis_index("c")
    pltpu.sync_copy(idx_hbm.at[wid], idx_v)

    def gather_descriptor(j, slot):
        return pltpu.make_async_copy(
            table_hbm.at[idx_v.at[j]], rows_v.at[slot], gsem[slot]
        )

    def write_descriptor(j, slot):
        return pltpu.make_async_copy(
            rows_v.at[slot], out_hbm.at[wid, j], wsem[slot]
        )

    for j in range(AHEAD):
        gather_descriptor(j, j).start()

    @pl.loop(0, nchunk, step=NSLOT)
    def _chunks(j0):
        for slot in range(NSLOT):
            j = j0 + slot

            @pl.when(j < nchunk)
            def _do_chunk():
                gather_descriptor(j, slot).wait()
                write_descriptor(j, slot).start()
                nslot = (slot + AHEAD) % NSLOT

                @pl.when(j + AHEAD < nchunk)
                def _prefetch():
                    @pl.when(j >= NSLOT - AHEAD)
                    def _free_slot():
                        write_descriptor(j + AHEAD - NSLOT, nslot).wait()

                    gather_descriptor(j + AHEAD, nslot).start()

    for j in range(nchunk):
        if j + NSLOT >= nchunk:
            write_descriptor(j, j % NSLOT).wait()


def kernel(tags, tag_embedding_weight):
    B_total = tags.shape[0] * tags.shape[1]
    D = tag_embedding_weight.shape[1]
    assert B_total % (NW * CHUNK) == 0
    nchunk = B_total // (NW * CHUNK)
    assert nchunk >= NSLOT > AHEAD

    idx = tags.reshape(NW, nchunk, CHUNK).astype(jnp.int32)

    mesh = plsc.VectorSubcoreMesh(
        core_axis_name="c", subcore_axis_name="s", num_cores=NC, num_subcores=NS
    )
    out = pl.kernel(
        functools.partial(_body, nchunk),
        out_type=jax.ShapeDtypeStruct((NW, nchunk, CHUNK, D), jnp.float32),
        mesh=mesh,
        scratch_types=[
            pltpu.VMEM((nchunk, CHUNK), jnp.int32),
            pltpu.VMEM((NSLOT, CHUNK, D), jnp.float32),
        ] + [pltpu.SemaphoreType.DMA] * (2 * NSLOT),
    )(tag_embedding_weight, idx)
    return out.reshape(tags.shape[0], tags.shape[1], D)
